# baseline (speedup 1.0000x reference)
import jax
import jax.numpy as jnp
import numpy as np
from jax import lax
from jax.experimental import pallas as pl
from jax.experimental.pallas import tpu as pltpu
from jax.experimental.pallas import tpu_sc as plsc

_NUM_CLASSES = 256
_EMB = 256
_IGNORE = 255

_NC = 2
_NS = 16
_NW = _NC * _NS

_W = 128
_HS = 64
_ROWS_PER_W = _HS // 4
_R_CHUNK = 8
_C_CHUNK = 32
_N_CCH = _EMB // _C_CHUNK
_N_PCH = _ROWS_PER_W // _R_CHUNK
_N_CHUNKS = _N_CCH * _N_PCH
_GROUPS = _R_CHUNK * _W // 16


def _sc_body(embs_hbm, lbl_hbm, tbl_hbm, out_hbm,
             tbl_v, lbl_v, acc_v, buf, stage, sem0, sem1):
    cid = lax.axis_index("c")
    sid = lax.axis_index("s")
    wid = sid * _NC + cid
    b = wid // 4
    h0 = (wid % 4) * _ROWS_PER_W

    pltpu.sync_copy(tbl_hbm, tbl_v)
    pltpu.sync_copy(lbl_hbm.at[b, pl.ds(h0, _ROWS_PER_W), :], lbl_v)

    def zacc(g, c):
        acc_v[g // 8, pl.ds((g % 8) * 16, 16)] = jnp.zeros((16,), jnp.float32)
        return c

    lax.fori_loop(0, _ROWS_PER_W * 8, zacc, 0)

    def chunk_src(t):
        cpart = t % _N_CCH
        ppart = t // _N_CCH
        return embs_hbm.at[b,
                           pl.ds(cpart * _C_CHUNK, _C_CHUNK),
                           pl.ds(h0 + ppart * _R_CHUNK, _R_CHUNK),
                           :]

    def compute_chunk(t, bufref):
        cbase = (t % _N_CCH) * _C_CHUNK
        rowb = (t // _N_CCH) * _R_CHUNK

        @plsc.parallel_loop(0, _GROUPS, unroll=2)
        def grp(g):
            hh = g // 8
            ws = (g % 8) * 16
            lvec = lbl_v[rowb + hh, pl.ds(ws, 16)]
            lbase = lvec + (cbase // 2) * _NUM_CLASSES
            parts = [jnp.zeros((16,), jnp.float32) for _ in range(4)]
            for k in range(_C_CHUNK // 2):
                w = plsc.load_gather(tbl_v, [lbase + k * _NUM_CLASSES])
                bf = plsc.bitcast(w, jnp.bfloat16)
                t0, t1 = plsc.unpack(bf, format=plsc.PackFormat.INTERLEAVED,
                                     preferred_element_type=jnp.float32)
                e0 = bufref[2 * k, hh, pl.ds(ws, 16)]
                e1 = bufref[2 * k + 1, hh, pl.ds(ws, 16)]
                parts[(2 * k) % 4] = parts[(2 * k) % 4] + jnp.abs(e0 - t0)
                parts[(2 * k + 1) % 4] = (parts[(2 * k + 1) % 4]
                                          + jnp.abs(e1 - t1))
            acc = (parts[0] + parts[1]) + (parts[2] + parts[3])
            acc_v[rowb + hh, pl.ds(ws, 16)] = (
                acc_v[rowb + hh, pl.ds(ws, 16)] + acc)

    pltpu.async_copy(chunk_src(0), buf.at[0], sem0)

    def pair(i, c):
        t0 = i * 2
        t1 = t0 + 1
        pltpu.make_async_copy(chunk_src(t0), buf.at[0], sem0).wait()
        pltpu.async_copy(chunk_src(t1), buf.at[1], sem1)
        compute_chunk(t0, buf.at[0])
        pltpu.make_async_copy(chunk_src(t1), buf.at[1], sem1).wait()

        @pl.when(i < _N_CHUNKS // 2 - 1)
        def _():
            pltpu.async_copy(chunk_src(t0 + 2), buf.at[0], sem0)

        compute_chunk(t1, buf.at[1])
        return c

    lax.fori_loop(0, _N_CHUNKS // 2, pair, 0)

    def fin(g, carry):
        s, cnt = carry
        row = g // 8
        ws = (g % 8) * 16
        lvec = lbl_v[row, pl.ds(ws, 16)]
        m = lvec != _IGNORE
        a = acc_v[row, pl.ds(ws, 16)]
        s = s + jnp.where(m, a, 0.0)
        cnt = cnt + jnp.where(m, 1.0, 0.0)
        return s, cnt

    zero = jnp.zeros((16,), jnp.float32)
    s, cnt = lax.fori_loop(0, _ROWS_PER_W * 8, fin, (zero, zero))
    stage[pl.ds(0, 16)] = s
    stage[pl.ds(16, 16)] = cnt
    pltpu.sync_copy(stage, out_hbm.at[wid])


_sc_loss = pl.kernel(
    _sc_body,
    out_type=jax.ShapeDtypeStruct((_NW, 32), jnp.float32),
    mesh=plsc.VectorSubcoreMesh(core_axis_name="c", subcore_axis_name="s",
                                num_cores=_NC, num_subcores=_NS),
    compiler_params=pltpu.CompilerParams(needs_layout_passes=False),
    scratch_types=[
        pltpu.VMEM((_EMB // 2 * _NUM_CLASSES,), jnp.int32),
        pltpu.VMEM((_ROWS_PER_W, _W), jnp.int32),
        pltpu.VMEM((_ROWS_PER_W, _W), jnp.float32),
        pltpu.VMEM((2, _C_CHUNK, _R_CHUNK, _W), jnp.float32),
        pltpu.VMEM((32,), jnp.float32),
        pltpu.SemaphoreType.DMA,
        pltpu.SemaphoreType.DMA,
    ],
)


_TC_RB = 32


def _tc_body(e_ref, l_ref, t_ref, o_ref, m_ref):
    px = _TC_RB * _W
    e2 = e_ref[0].reshape(_EMB, px)
    lab = l_ref[0].reshape(1, px)
    cls = jax.lax.broadcasted_iota(jnp.int32, (_NUM_CLASSES, px), 0)
    onehot = (cls == lab).astype(jnp.bfloat16)
    g = jax.lax.dot_general(t_ref[...], onehot, (((1,), (0,)), ((), ())),
                            preferred_element_type=jnp.float32)
    mask = (lab != _IGNORE).astype(jnp.float32)
    d = jnp.abs(e2 - g) * mask

    def fold(x):
        acc = x[0:8]
        for r in range(8, _TC_RB, 8):
            acc = acc + x[r:r + 8]
        return acc

    ps = fold(jnp.sum(d, axis=0).reshape(_TC_RB, _W))
    pm = fold(mask.reshape(1, px).reshape(_TC_RB, _W))

    first = (pl.program_id(0) == 0) & (pl.program_id(1) == 0)

    @pl.when(first)
    def _():
        o_ref[...] = ps
        m_ref[...] = pm

    @pl.when(jnp.logical_not(first))
    def _():
        o_ref[...] = o_ref[...] + ps
        m_ref[...] = m_ref[...] + pm


_N_TC_HB = (128 - _HS) // _TC_RB

_tc_loss = pl.pallas_call(
    _tc_body,
    grid=(8, _N_TC_HB),
    in_specs=[
        pl.BlockSpec((1, _EMB, _TC_RB, _W),
                     lambda b, j: (b, 0, _HS // _TC_RB + j, 0)),
        pl.BlockSpec((1, _TC_RB, _W), lambda b, j: (b, _HS // _TC_RB + j, 0)),
        pl.BlockSpec((_EMB, _NUM_CLASSES), lambda b, j: (0, 0)),
    ],
    out_specs=[
        pl.BlockSpec((8, _W), lambda b, j: (0, 0)),
        pl.BlockSpec((8, _W), lambda b, j: (0, 0)),
    ],
    out_shape=[
        jax.ShapeDtypeStruct((8, _W), jnp.float32),
        jax.ShapeDtypeStruct((8, _W), jnp.float32),
    ],
)


def kernel(embs, labels, idx_to_vec):
    B, C, H, W = embs.shape
    lbl = labels.astype(jnp.int32)
    tb = idx_to_vec.astype(jnp.bfloat16)
    packed = jax.lax.bitcast_convert_type(
        tb.reshape(_NUM_CLASSES, _EMB // 2, 2), jnp.int32)
    tbl = packed.T.reshape(-1)
    sc_out = _sc_loss(embs, lbl, tbl)
    tc_sum, tc_msk = _tc_loss(embs, lbl, tb.T)
    psum = jnp.sum(sc_out[:, :16]) + jnp.sum(tc_sum)
    pcnt = jnp.sum(sc_out[:, 16:]) + jnp.sum(tc_msk)
    return psum / (pcnt * np.float32(C))


# speedup vs baseline: 1.0075x; 1.0075x over previous
import jax
import jax.numpy as jnp
import numpy as np
from jax import lax
from jax.experimental import pallas as pl
from jax.experimental.pallas import tpu as pltpu
from jax.experimental.pallas import tpu_sc as plsc

_NUM_CLASSES = 256
_EMB = 256
_IGNORE = 255

_NC = 2
_NS = 16
_NW = _NC * _NS

_W = 128
_HS = 64
_ROWS_PER_W = _HS // 4
_R_CHUNK = 8
_C_CHUNK = 32
_N_CCH = _EMB // _C_CHUNK
_N_PCH = _ROWS_PER_W // _R_CHUNK
_N_CHUNKS = _N_CCH * _N_PCH
_GROUPS = _R_CHUNK * _W // 16


def _sc_body(embs_hbm, lbl_hbm, tbl_hbm, out_hbm,
             tbl_v, lbl_v, acc_v, buf, stage, sem0, sem1):
    cid = lax.axis_index("c")
    sid = lax.axis_index("s")
    wid = sid * _NC + cid
    b = wid // 4
    h0 = (wid % 4) * _ROWS_PER_W

    pltpu.sync_copy(tbl_hbm, tbl_v)
    pltpu.sync_copy(lbl_hbm.at[b, pl.ds(h0, _ROWS_PER_W), :], lbl_v)

    def zacc(g, c):
        acc_v[g // 8, pl.ds((g % 8) * 16, 16)] = jnp.zeros((16,), jnp.float32)
        return c

    lax.fori_loop(0, _ROWS_PER_W * 8, zacc, 0)

    def chunk_src(t):
        cpart = t % _N_CCH
        ppart = t // _N_CCH
        return embs_hbm.at[b,
                           pl.ds(cpart * _C_CHUNK, _C_CHUNK),
                           pl.ds(h0 + ppart * _R_CHUNK, _R_CHUNK),
                           :]

    def compute_chunk(t, bufref):
        cbase = (t % _N_CCH) * _C_CHUNK
        rowb = (t // _N_CCH) * _R_CHUNK

        @plsc.parallel_loop(0, _GROUPS, unroll=2)
        def grp(g):
            hh = g // 8
            ws = (g % 8) * 16
            lvec = lbl_v[rowb + hh, pl.ds(ws, 16)]
            lbase = lvec + (cbase // 2) * _NUM_CLASSES
            parts = [jnp.zeros((16,), jnp.float32) for _ in range(4)]
            for k in range(_C_CHUNK // 2):
                w = plsc.load_gather(tbl_v, [lbase + k * _NUM_CLASSES])
                bf = plsc.bitcast(w, jnp.bfloat16)
                t0, t1 = plsc.unpack(bf, format=plsc.PackFormat.INTERLEAVED,
                                     preferred_element_type=jnp.float32)
                e0 = bufref[2 * k, hh, pl.ds(ws, 16)]
                e1 = bufref[2 * k + 1, hh, pl.ds(ws, 16)]
                parts[(2 * k) % 4] = parts[(2 * k) % 4] + jnp.abs(e0 - t0)
                parts[(2 * k + 1) % 4] = (parts[(2 * k + 1) % 4]
                                          + jnp.abs(e1 - t1))
            acc = (parts[0] + parts[1]) + (parts[2] + parts[3])
            acc_v[rowb + hh, pl.ds(ws, 16)] = (
                acc_v[rowb + hh, pl.ds(ws, 16)] + acc)

    pltpu.async_copy(chunk_src(0), buf.at[0], sem0)

    def pair(i, c):
        t0 = i * 2
        t1 = t0 + 1
        pltpu.make_async_copy(chunk_src(t0), buf.at[0], sem0).wait()
        pltpu.async_copy(chunk_src(t1), buf.at[1], sem1)
        compute_chunk(t0, buf.at[0])
        pltpu.make_async_copy(chunk_src(t1), buf.at[1], sem1).wait()

        @pl.when(i < _N_CHUNKS // 2 - 1)
        def _():
            pltpu.async_copy(chunk_src(t0 + 2), buf.at[0], sem0)

        compute_chunk(t1, buf.at[1])
        return c

    lax.fori_loop(0, _N_CHUNKS // 2, pair, 0)

    def fin(g, carry):
        s, cnt = carry
        row = g // 8
        ws = (g % 8) * 16
        lvec = lbl_v[row, pl.ds(ws, 16)]
        m = lvec != _IGNORE
        a = acc_v[row, pl.ds(ws, 16)]
        s = s + jnp.where(m, a, 0.0)
        cnt = cnt + jnp.where(m, 1.0, 0.0)
        return s, cnt

    zero = jnp.zeros((16,), jnp.float32)
    s, cnt = lax.fori_loop(0, _ROWS_PER_W * 8, fin, (zero, zero))
    stage[pl.ds(0, 16)] = s
    stage[pl.ds(16, 16)] = cnt
    pltpu.sync_copy(stage, out_hbm.at[wid])


_sc_loss = pl.kernel(
    _sc_body,
    out_type=jax.ShapeDtypeStruct((_NW, 32), jnp.float32),
    mesh=plsc.VectorSubcoreMesh(core_axis_name="c", subcore_axis_name="s",
                                num_cores=_NC, num_subcores=_NS),
    compiler_params=pltpu.CompilerParams(needs_layout_passes=False),
    scratch_types=[
        pltpu.VMEM((_EMB // 2 * _NUM_CLASSES,), jnp.int32),
        pltpu.VMEM((_ROWS_PER_W, _W), jnp.int32),
        pltpu.VMEM((_ROWS_PER_W, _W), jnp.float32),
        pltpu.VMEM((2, _C_CHUNK, _R_CHUNK, _W), jnp.float32),
        pltpu.VMEM((32,), jnp.float32),
        pltpu.SemaphoreType.DMA,
        pltpu.SemaphoreType.DMA,
    ],
)


_TC_RB = 16


def _tc_body(e_ref, l_ref, t_ref, o_ref, m_ref):
    px = _TC_RB * _W
    e2 = e_ref[0].reshape(_EMB, px)
    lab = l_ref[0].reshape(1, px)
    cls = jax.lax.broadcasted_iota(jnp.int32, (_NUM_CLASSES, px), 0)
    onehot = (cls == lab).astype(jnp.bfloat16)
    g = jax.lax.dot_general(t_ref[...], onehot, (((1,), (0,)), ((), ())),
                            preferred_element_type=jnp.float32)
    mask = (lab != _IGNORE).astype(jnp.float32)
    d = jnp.abs(e2 - g) * mask

    def fold(x):
        acc = x[0:8]
        for r in range(8, _TC_RB, 8):
            acc = acc + x[r:r + 8]
        return acc

    ps = fold(jnp.sum(d, axis=0).reshape(_TC_RB, _W))
    pm = fold(mask.reshape(1, px).reshape(_TC_RB, _W))

    first = (pl.program_id(0) == 0) & (pl.program_id(1) == 0)

    @pl.when(first)
    def _():
        o_ref[...] = ps
        m_ref[...] = pm

    @pl.when(jnp.logical_not(first))
    def _():
        o_ref[...] = o_ref[...] + ps
        m_ref[...] = m_ref[...] + pm


_N_TC_HB = (128 - _HS) // _TC_RB

_tc_loss = pl.pallas_call(
    _tc_body,
    grid=(8, _N_TC_HB),
    in_specs=[
        pl.BlockSpec((1, _EMB, _TC_RB, _W),
                     lambda b, j: (b, 0, _HS // _TC_RB + j, 0)),
        pl.BlockSpec((1, _TC_RB, _W), lambda b, j: (b, _HS // _TC_RB + j, 0)),
        pl.BlockSpec((_EMB, _NUM_CLASSES), lambda b, j: (0, 0)),
    ],
    out_specs=[
        pl.BlockSpec((8, _W), lambda b, j: (0, 0)),
        pl.BlockSpec((8, _W), lambda b, j: (0, 0)),
    ],
    out_shape=[
        jax.ShapeDtypeStruct((8, _W), jnp.float32),
        jax.ShapeDtypeStruct((8, _W), jnp.float32),
    ],
)


def kernel(embs, labels, idx_to_vec):
    B, C, H, W = embs.shape
    lbl = labels.astype(jnp.int32)
    tb = idx_to_vec.astype(jnp.bfloat16)
    packed = jax.lax.bitcast_convert_type(
        tb.reshape(_NUM_CLASSES, _EMB // 2, 2), jnp.int32)
    tbl = packed.T.reshape(-1)
    sc_out = _sc_loss(embs, lbl, tbl)
    tc_sum, tc_msk = _tc_loss(embs, lbl, tb.T)
    psum = jnp.sum(sc_out[:, :16]) + jnp.sum(tc_sum)
    pcnt = jnp.sum(sc_out[:, 16:]) + jnp.sum(tc_msk)
    return psum / (pcnt * np.float32(C))
